# Initial kernel scaffold; baseline (speedup 1.0000x reference)
import jax
import jax.numpy as jnp
from jax.experimental import pallas as pl


def kernel(site_energy, batch, mean, std):
    raise NotImplementedError("write your pallas kernel here")


# speedup vs baseline: 20.8991x; 20.8991x over previous
import functools

import jax
import jax.numpy as jnp
from jax import lax
from jax.experimental import pallas as pl
from jax.experimental.pallas import tpu as pltpu
from jax.experimental.pallas import tpu_sc as plsc

N = 3_200_000
S = 100_000
NW = 32
W = N // NW
C = 2_000
NCH = W // C
VPC = C // 16
UNR = 5
SP = 100_096
P = S + 128
ZUNR = 6

_mesh = plsc.VectorSubcoreMesh(core_axis_name="c", subcore_axis_name="s")


@functools.partial(
    pl.kernel,
    out_type=jax.ShapeDtypeStruct((2, SP), jnp.float32),
    mesh=_mesh,
    scratch_types=[
        pltpu.VMEM((P,), jnp.float32),
        pltpu.VMEM((C,), jnp.float32),
        pltpu.VMEM((C,), jnp.float32),
        pltpu.VMEM((C,), jnp.int32),
        pltpu.VMEM((C,), jnp.int32),
        pltpu.VMEM((32,), jnp.float32),
        pltpu.VMEM((128,), jnp.int32),
        pltpu.VMEM_SHARED((P,), jnp.float32),
        pltpu.SemaphoreType.DMA,
        pltpu.SemaphoreType.DMA,
        pltpu.SemaphoreType.DMA,
        pltpu.SemaphoreType.DMA,
    ],
    compiler_params=pltpu.CompilerParams(needs_layout_passes=False),
)
def _sc_segment_sum(se_hbm, b_hbm, ms_hbm, out_hbm,
                    acc, d0, d1, i0, i1, par, sidx, acc_sh,
                    sd0, si0, sd1, si1):
    cid = lax.axis_index("c")
    sid = lax.axis_index("s")
    wid = cid * 16 + sid
    base = wid * W

    zeros16 = jnp.zeros((16,), jnp.float32)

    def zbody(i, carry):
        for u in range(ZUNR):
            acc[pl.ds((i * ZUNR + u) * 16, 16)] = zeros16
        return carry
    lax.fori_loop(0, P // 16 // ZUNR, zbody, 0)

    @pl.when(sid == 0)
    def _():
        pltpu.sync_copy(acc, acc_sh)
    plsc.subcore_barrier()

    pltpu.sync_copy(ms_hbm, par)
    mean_v = par[pl.ds(0, 16)]
    std_v = par[pl.ds(16, 16)]

    dat = (d0, d1)
    idx = (i0, i1)
    sems_d = (sd0, sd1)
    sems_i = (si0, si1)
    descs = [None, None]

    def start(ch, b):
        off = base + ch * C
        dd = pltpu.async_copy(se_hbm.at[pl.ds(off, C)], dat[b], sems_d[b])
        di = pltpu.async_copy(b_hbm.at[pl.ds(off, C)], idx[b], sems_i[b])
        descs[b] = (dd, di)

    def process(dref, iref):
        def body(i, carry):
            for u in range(UNR):
                sl = pl.ds((i * UNR + u) * 16, 16)
                seg = iref[sl]
                val = dref[sl] * std_v + mean_v
                plsc.addupdate_scatter(acc, [seg], val)
            return carry
        lax.fori_loop(0, VPC // UNR, body, 0)

    start(0, 0)
    first_seg = jnp.int32(0)
    for ch in range(NCH):
        b = ch & 1
        descs[b][0].wait()
        descs[b][1].wait()
        if ch + 1 < NCH:
            start(ch + 1, 1 - b)
        if ch == 0:
            first_seg = idx[0][pl.ds(0, 16)][0]
        process(dat[b], idx[b])
    last_seg = idx[(NCH - 1) & 1][pl.ds(C - 16, 16)][15]

    first_al = (first_seg // 8) * 8
    nch = (last_seg - first_al) // 128 + 1
    iota = lax.iota(jnp.int32, 16)

    def cbody(k, carry):
        bs = first_al + k * 128
        for j in range(8):
            sidx[pl.ds(j * 16, 16)] = iota + (bs + j * 16)
        pltpu.sync_copy(acc.at[pl.ds(bs, 128)], acc_sh.at[sidx], add=True)
        return carry
    lax.fori_loop(0, nch, cbody, 0)

    plsc.subcore_barrier()

    @pl.when(sid == 0)
    def _():
        pltpu.sync_copy(acc_sh.at[pl.ds(0, SP)], out_hbm.at[cid])


def _combine_body(p_ref, o_ref):
    o_ref[...] = p_ref[0:1, :] + p_ref[1:2, :]


def kernel(site_energy, batch, mean, std):
    b32 = batch.astype(jnp.int32)
    ms = jnp.concatenate([
        jnp.full((16,), mean, jnp.float32),
        jnp.full((16,), std, jnp.float32),
    ])
    partials = _sc_segment_sum(site_energy, b32, ms)
    out = pl.pallas_call(
        _combine_body,
        out_shape=jax.ShapeDtypeStruct((1, SP), jnp.float32),
    )(partials)
    return out.reshape(SP)[:S]
